# baseline (speedup 1.0000x reference)
import functools

import jax
import jax.numpy as jnp
from jax import lax
from jax.experimental import pallas as pl
from jax.experimental.pallas import tpu as pltpu
from jax.experimental.pallas import tpu_sc as plsc

_B, _N, _CIN = 8, 4096, 64
_S, _K = 1024, 32
_EPS = 1e-5
_M = float(_B * _S * _K)


def _fps_body(xyz_ref, nxyz_ref):
    x = xyz_ref[:, 0, :]
    y = xyz_ref[:, 1, :]
    z = xyz_ref[:, 2, :]
    iota_n = lax.broadcasted_iota(jnp.int32, (_B, _N), 1)
    iota_s = lax.broadcasted_iota(jnp.int32, (_B, _S), 1)

    def step(i, carry):
        dist, far = carry
        sel = iota_n == far
        cx = jnp.sum(jnp.where(sel, x, 0.0), axis=1, keepdims=True)
        cy = jnp.sum(jnp.where(sel, y, 0.0), axis=1, keepdims=True)
        cz = jnp.sum(jnp.where(sel, z, 0.0), axis=1, keepdims=True)
        here = iota_s == i
        nxyz_ref[:, 0, :] = jnp.where(here, cx, nxyz_ref[:, 0, :])
        nxyz_ref[:, 1, :] = jnp.where(here, cy, nxyz_ref[:, 1, :])
        nxyz_ref[:, 2, :] = jnp.where(here, cz, nxyz_ref[:, 2, :])
        dx = x - cx
        dy = y - cy
        dz = z - cz
        d = (dx * dx + dy * dy) + dz * dz
        dist = jnp.minimum(dist, d)
        maxv = jnp.max(dist, axis=1, keepdims=True)
        far = jnp.min(jnp.where(dist == maxv, iota_n, _N), axis=1, keepdims=True)
        return dist, far.astype(jnp.int32)

    init = (
        jnp.full((_B, _N), 1e10, jnp.float32),
        jnp.zeros((_B, 1), jnp.int32),
    )
    lax.fori_loop(0, _S, step, init)


def _fps(xyz):
    return pl.pallas_call(
        _fps_body,
        out_shape=jax.ShapeDtypeStruct((_B, 3, _S), jnp.float32),
    )(xyz)


def _knn_body(nx_ref, xyz_ref, idx_ref, *, b0):
    b = pl.program_id(0) + b0
    q3 = nx_ref[0]
    x3 = xyz_ref[0]
    qb = q3.shape[1]
    pn2 = jnp.sum(x3 * x3, axis=0, keepdims=True)
    qp = lax.dot_general(q3, x3, (((0,), (0,)), ((), ())),
                         preferred_element_type=jnp.float32)
    dist = pn2 - 2.0 * qp
    iota_n = lax.broadcasted_iota(jnp.int32, (qb, _N), 1)
    iota_k = lax.broadcasted_iota(jnp.int32, (qb, _K), 1)
    base = b * _N

    def step(k, dist):
        m = jnp.min(dist, axis=1, keepdims=True)
        idx = jnp.min(jnp.where(dist == m, iota_n, _N), axis=1, keepdims=True)
        idx_ref[0] = jnp.where(iota_k == k, idx + base, idx_ref[0])
        return jnp.where(iota_n == idx, 1e30, dist)

    lax.fori_loop(0, _K, step, dist)


def _knn(new_xyz, xyz, b0, nb):
    qblk = 128
    return pl.pallas_call(
        functools.partial(_knn_body, b0=b0),
        grid=(nb, _S // qblk),
        in_specs=[
            pl.BlockSpec((1, 3, qblk), lambda b, s: (b, 0, s)),
            pl.BlockSpec((1, 3, _N), lambda b, s: (b, 0, 0)),
        ],
        out_specs=pl.BlockSpec((1, qblk, _K), lambda b, s: (b, s, 0)),
        out_shape=jax.ShapeDtypeStruct((nb, _S, _K), jnp.int32),
    )(new_xyz, xyz)


def _proj_body(xyz_ref, pts_ref, w_ref, z_ref):
    w1x = w_ref[:, 0:3]
    w1p = w_ref[:, 3:]
    zx = lax.dot_general(xyz_ref[0], w1x, (((0,), (1,)), ((), ())),
                         preferred_element_type=jnp.float32)
    zp = lax.dot_general(pts_ref[0], w1p, (((0,), (1,)), ((), ())),
                         preferred_element_type=jnp.float32)
    z_ref[0] = zx + zp


def _project(xyz, points, w1):
    nblk = 512
    return pl.pallas_call(
        _proj_body,
        grid=(_B, _N // nblk),
        in_specs=[
            pl.BlockSpec((1, 3, nblk), lambda b, n: (b, 0, n)),
            pl.BlockSpec((1, _CIN, nblk), lambda b, n: (b, 0, n)),
            pl.BlockSpec((64, _CIN + 3), lambda b, n: (0, 0)),
        ],
        out_specs=pl.BlockSpec((1, nblk, 64), lambda b, n: (b, n, 0)),
        out_shape=jax.ShapeDtypeStruct((_B, _N, 64), jnp.float32),
    )(xyz, points, w1)


def _gather_sc(table, idx):
    rows = idx.shape[0]
    d = table.shape[1]
    info = plsc.get_sparse_core_info()
    nw = info.num_cores * info.num_subcores
    chunk = 128
    per_w = rows // nw
    nchunk = per_w // chunk

    mesh = plsc.VectorSubcoreMesh(core_axis_name="c", subcore_axis_name="s")

    @functools.partial(
        pl.kernel,
        mesh=mesh,
        compiler_params=pltpu.CompilerParams(use_tc_tiling_on_sc=False),
        out_type=jax.ShapeDtypeStruct((rows, d), jnp.float32),
        scratch_types=[
            pltpu.VMEM((chunk,), jnp.int32),
            pltpu.VMEM((chunk, d), jnp.float32),
            pltpu.SemaphoreType.DMA,
        ],
    )
    def k(table_hbm, idx_hbm, out_hbm, idx_v, rows_v, sem):
        wid = lax.axis_index("s") * info.num_cores + lax.axis_index("c")
        base = wid * per_w

        def body(j, _):
            off = base + j * chunk
            pltpu.sync_copy(idx_hbm.at[pl.ds(off, chunk)], idx_v)
            pltpu.async_copy(table_hbm.at[idx_v], rows_v, sem).wait()
            pltpu.sync_copy(rows_v, out_hbm.at[pl.ds(off, chunk)])
            return 0

        lax.fori_loop(0, nchunk, body, 0)

    return k(table, idx)


def _bn_coefs(s_ref, q_ref, g_ref, b_ref):
    mean = s_ref[...] / _M
    var = q_ref[...] / _M - mean * mean
    scale = g_ref[...] / jnp.sqrt(var + _EPS)
    shift = b_ref[...] - mean * scale
    return scale, shift


def _acc_stats(first, y, s_ref, q_ref, width):
    psum = jnp.sum(y, axis=0).reshape(1, width)
    pq = jnp.sum(y * y, axis=0).reshape(1, width)

    @pl.when(first)
    def _():
        s_ref[...] = jnp.zeros_like(s_ref)
        q_ref[...] = jnp.zeros_like(q_ref)

    s_ref[...] += psum
    q_ref[...] += pq


def _first(b, sb):
    return jnp.logical_and(b == 0, sb == 0)


def _stats1_body(zg_ref, nx_ref, w_ref, c1_ref, s_ref, q_ref):
    w1x = w_ref[:, 0:3]
    c1 = lax.dot_general(nx_ref[0], w1x, (((0,), (1,)), ((), ())),
                         preferred_element_type=jnp.float32)
    c1_ref[0] = c1
    sblk = c1.shape[0]
    zg = zg_ref[0].reshape(sblk, _K, 64)
    y1 = (zg - c1[:, None, :]).reshape(sblk * _K, 64)
    _acc_stats(_first(pl.program_id(0), pl.program_id(1)), y1, s_ref, q_ref, 64)


def _stats1(zg3, new_xyz, w1):
    sblk = 128
    return pl.pallas_call(
        _stats1_body,
        grid=(_B, _S // sblk),
        in_specs=[
            pl.BlockSpec((1, sblk * _K, 64), lambda b, s: (b, s, 0)),
            pl.BlockSpec((1, 3, sblk), lambda b, s: (b, 0, s)),
            pl.BlockSpec((64, _CIN + 3), lambda b, s: (0, 0)),
        ],
        out_specs=[
            pl.BlockSpec((1, sblk, 64), lambda b, s: (b, s, 0)),
            pl.BlockSpec((1, 64), lambda b, s: (0, 0)),
            pl.BlockSpec((1, 64), lambda b, s: (0, 0)),
        ],
        out_shape=[
            jax.ShapeDtypeStruct((_B, _S, 64), jnp.float32),
            jax.ShapeDtypeStruct((1, 64), jnp.float32),
            jax.ShapeDtypeStruct((1, 64), jnp.float32),
        ],
    )(zg3, new_xyz, w1)


def _y2_of(zg_ref, c1_ref, s1_ref, q1_ref, g1_ref, b1_ref, w2_ref):
    scale, shift = _bn_coefs(s1_ref, q1_ref, g1_ref, b1_ref)
    c1 = c1_ref[0]
    sblk = c1.shape[0]
    zg = zg_ref[0].reshape(sblk, _K, 64)
    y1 = zg - c1[:, None, :]
    y1n = jnp.maximum(y1 * scale.reshape(1, 1, 64) + shift.reshape(1, 1, 64), 0.0)
    return lax.dot_general(y1n.reshape(sblk * _K, 64), w2_ref[...],
                           (((1,), (1,)), ((), ())),
                           preferred_element_type=jnp.float32)


def _mlp_specs(sblk, extra):
    return [
        pl.BlockSpec((1, sblk * _K, 64), lambda b, s: (b, s, 0)),
        pl.BlockSpec((1, sblk, 64), lambda b, s: (b, s, 0)),
        pl.BlockSpec((1, 64), lambda b, s: (0, 0)),
        pl.BlockSpec((1, 64), lambda b, s: (0, 0)),
        pl.BlockSpec((1, 64), lambda b, s: (0, 0)),
        pl.BlockSpec((1, 64), lambda b, s: (0, 0)),
        pl.BlockSpec((128, 64), lambda b, s: (0, 0)),
    ] + extra


def _l2s_body(zg_ref, c1_ref, s1_ref, q1_ref, g1_ref, b1_ref, w2_ref,
              s2_ref, q2_ref):
    y2 = _y2_of(zg_ref, c1_ref, s1_ref, q1_ref, g1_ref, b1_ref, w2_ref)
    _acc_stats(_first(pl.program_id(0), pl.program_id(1)), y2, s2_ref, q2_ref, 128)


def _l2_stats(zg3, c1, s1, q1, g1, b1, w2):
    sblk = 32
    return pl.pallas_call(
        _l2s_body,
        grid=(_B, _S // sblk),
        in_specs=_mlp_specs(sblk, []),
        out_specs=[
            pl.BlockSpec((1, 128), lambda b, s: (0, 0)),
            pl.BlockSpec((1, 128), lambda b, s: (0, 0)),
        ],
        out_shape=[
            jax.ShapeDtypeStruct((1, 128), jnp.float32),
            jax.ShapeDtypeStruct((1, 128), jnp.float32),
        ],
    )(zg3, c1, s1, q1, g1, b1, w2)


def _l3s_body(zg_ref, c1_ref, s1_ref, q1_ref, g1_ref, b1_ref, w2_ref,
              s2_ref, q2_ref, g2_ref, b2_ref, w3_ref, s3_ref, q3_ref):
    y2 = _y2_of(zg_ref, c1_ref, s1_ref, q1_ref, g1_ref, b1_ref, w2_ref)
    scale2, shift2 = _bn_coefs(s2_ref, q2_ref, g2_ref, b2_ref)
    y2n = jnp.maximum(y2 * scale2 + shift2, 0.0)
    y3 = lax.dot_general(y2n, w3_ref[...], (((1,), (1,)), ((), ())),
                         preferred_element_type=jnp.float32)
    _acc_stats(_first(pl.program_id(0), pl.program_id(1)), y3, s3_ref, q3_ref, 256)


def _l3_stats(zg3, c1, s1, q1, g1, b1, w2, s2, q2, g2, b2, w3):
    sblk = 32
    extra = [
        pl.BlockSpec((1, 128), lambda b, s: (0, 0)),
        pl.BlockSpec((1, 128), lambda b, s: (0, 0)),
        pl.BlockSpec((1, 128), lambda b, s: (0, 0)),
        pl.BlockSpec((1, 128), lambda b, s: (0, 0)),
        pl.BlockSpec((256, 128), lambda b, s: (0, 0)),
    ]
    return pl.pallas_call(
        _l3s_body,
        grid=(_B, _S // sblk),
        in_specs=_mlp_specs(sblk, extra),
        out_specs=[
            pl.BlockSpec((1, 256), lambda b, s: (0, 0)),
            pl.BlockSpec((1, 256), lambda b, s: (0, 0)),
        ],
        out_shape=[
            jax.ShapeDtypeStruct((1, 256), jnp.float32),
            jax.ShapeDtypeStruct((1, 256), jnp.float32),
        ],
    )(zg3, c1, s1, q1, g1, b1, w2, s2, q2, g2, b2, w3)


def _final_body(zg_ref, c1_ref, s1_ref, q1_ref, g1_ref, b1_ref, w2_ref,
                s2_ref, q2_ref, g2_ref, b2_ref, w3_ref,
                s3_ref, q3_ref, g3_ref, b3_ref, out_ref):
    y2 = _y2_of(zg_ref, c1_ref, s1_ref, q1_ref, g1_ref, b1_ref, w2_ref)
    scale2, shift2 = _bn_coefs(s2_ref, q2_ref, g2_ref, b2_ref)
    y2n = jnp.maximum(y2 * scale2 + shift2, 0.0)
    y3 = lax.dot_general(y2n, w3_ref[...], (((1,), (1,)), ((), ())),
                         preferred_element_type=jnp.float32)
    scale3, shift3 = _bn_coefs(s3_ref, q3_ref, g3_ref, b3_ref)
    y3n = jnp.maximum(y3 * scale3 + shift3, 0.0)
    sblk = y3n.shape[0] // _K
    out_ref[0] = jnp.max(y3n.reshape(sblk, _K, 256), axis=1)


def _final(zg3, c1, s1, q1, g1, b1, w2, s2, q2, g2, b2, w3, s3, q3, g3, b3):
    sblk = 32
    extra = [
        pl.BlockSpec((1, 128), lambda b, s: (0, 0)),
        pl.BlockSpec((1, 128), lambda b, s: (0, 0)),
        pl.BlockSpec((1, 128), lambda b, s: (0, 0)),
        pl.BlockSpec((1, 128), lambda b, s: (0, 0)),
        pl.BlockSpec((256, 128), lambda b, s: (0, 0)),
        pl.BlockSpec((1, 256), lambda b, s: (0, 0)),
        pl.BlockSpec((1, 256), lambda b, s: (0, 0)),
        pl.BlockSpec((1, 256), lambda b, s: (0, 0)),
        pl.BlockSpec((1, 256), lambda b, s: (0, 0)),
    ]
    return pl.pallas_call(
        _final_body,
        grid=(_B, _S // sblk),
        in_specs=_mlp_specs(sblk, extra),
        out_specs=pl.BlockSpec((1, sblk, 256), lambda b, s: (b, s, 0)),
        out_shape=jax.ShapeDtypeStruct((_B, _S, 256), jnp.float32),
    )(zg3, c1, s1, q1, g1, b1, w2, s2, q2, g2, b2, w3, s3, q3, g3, b3)


def kernel(xyz, points, W1, g1, b1, W2, g2, b2, W3, g3, b3):
    new_xyz = _fps(xyz)
    z = _project(xyz, points, W1)
    table = z.reshape(_B * _N, 64)

    nb = 2
    zg_parts = []
    for g in range(_B // nb):
        b0 = g * nb
        idx_g = _knn(new_xyz[b0:b0 + nb], xyz[b0:b0 + nb], b0, nb)
        zg_parts.append(_gather_sc(table, idx_g.reshape(-1)))
    zg3 = jnp.concatenate(zg_parts, axis=0).reshape(_B, _S * _K, 64)

    g1r, b1r = g1.reshape(1, 64), b1.reshape(1, 64)
    g2r, b2r = g2.reshape(1, 128), b2.reshape(1, 128)
    g3r, b3r = g3.reshape(1, 256), b3.reshape(1, 256)

    c1, s1, q1 = _stats1(zg3, new_xyz, W1)
    s2, q2 = _l2_stats(zg3, c1, s1, q1, g1r, b1r, W2)
    s3, q3 = _l3_stats(zg3, c1, s1, q1, g1r, b1r, W2, s2, q2, g2r, b2r, W3)
    out = _final(zg3, c1, s1, q1, g1r, b1r, W2, s2, q2, g2r, b2r, W3,
                 s3, q3, g3r, b3r)

    return (new_xyz, jnp.transpose(out, (0, 2, 1)))


# speedup vs baseline: 1.1649x; 1.1649x over previous
import functools

import jax
import jax.numpy as jnp
from jax import lax
from jax.experimental import pallas as pl
from jax.experimental.pallas import tpu as pltpu
from jax.experimental.pallas import tpu_sc as plsc

_B, _N, _CIN = 8, 4096, 64
_S, _K = 1024, 32
_EPS = 1e-5
_M = float(_B * _S * _K)


def _fps_body(xyz_ref, nxyz_ref):
    x = xyz_ref[:, 0, :]
    y = xyz_ref[:, 1, :]
    z = xyz_ref[:, 2, :]
    iota_n = lax.broadcasted_iota(jnp.int32, (_B, _N), 1)
    iota_s = lax.broadcasted_iota(jnp.int32, (_B, _S), 1)

    def step(i, carry):
        dist, far = carry
        sel = iota_n == far
        cx = jnp.sum(jnp.where(sel, x, 0.0), axis=1, keepdims=True)
        cy = jnp.sum(jnp.where(sel, y, 0.0), axis=1, keepdims=True)
        cz = jnp.sum(jnp.where(sel, z, 0.0), axis=1, keepdims=True)
        here = iota_s == i
        nxyz_ref[:, 0, :] = jnp.where(here, cx, nxyz_ref[:, 0, :])
        nxyz_ref[:, 1, :] = jnp.where(here, cy, nxyz_ref[:, 1, :])
        nxyz_ref[:, 2, :] = jnp.where(here, cz, nxyz_ref[:, 2, :])
        dx = x - cx
        dy = y - cy
        dz = z - cz
        d = (dx * dx + dy * dy) + dz * dz
        dist = jnp.minimum(dist, d)
        maxv = jnp.max(dist, axis=1, keepdims=True)
        far = jnp.min(jnp.where(dist == maxv, iota_n, _N), axis=1, keepdims=True)
        return dist, far.astype(jnp.int32)

    init = (
        jnp.full((_B, _N), 1e10, jnp.float32),
        jnp.zeros((_B, 1), jnp.int32),
    )
    lax.fori_loop(0, _S, step, init)


def _fps(xyz):
    return pl.pallas_call(
        _fps_body,
        out_shape=jax.ShapeDtypeStruct((_B, 3, _S), jnp.float32),
    )(xyz)


_GW = 32
_NG = _N // _GW


def _knn_body(nx_ref, xyz_ref, idx_ref, gid_ref):
    b = pl.program_id(0)
    q3 = nx_ref[0]
    x3 = xyz_ref[0]
    qb = q3.shape[1]
    pn2 = jnp.sum(x3 * x3, axis=0, keepdims=True)
    qp = lax.dot_general(q3, x3, (((0,), (0,)), ((), ())),
                         preferred_element_type=jnp.float32)
    dist = pn2 - 2.0 * qp

    d3 = dist.reshape(qb, _NG, _GW)
    gmin = jnp.min(d3, axis=2)
    iota_g = lax.broadcasted_iota(jnp.int32, (qb, _NG), 1)
    iota_c = lax.broadcasted_iota(jnp.int32, (qb, _K), 1)

    def gstep(k, gm):
        m = jnp.min(gm, axis=1, keepdims=True)
        g = jnp.min(jnp.where(gm == m, iota_g, _NG), axis=1, keepdims=True)
        gid_ref[...] = jnp.where(iota_c == k, g, gid_ref[...])
        return jnp.where(iota_g == g, 1e30, gm)

    lax.fori_loop(0, _K, gstep, gmin)
    gid = gid_ref[...]

    onehot = (gid[:, :, None] == iota_g[:, None, :]).astype(jnp.float32)
    cand = lax.dot_general(onehot, d3, (((2,), (1,)), ((0,), (0,))),
                           preferred_element_type=jnp.float32)
    cand = cand.reshape(qb, _K * _GW)
    iota_l = lax.broadcasted_iota(jnp.int32, (qb, _K, _GW), 2)
    ncand = (gid[:, :, None] * _GW + iota_l).reshape(qb, _K * _GW)
    base = b * _N
    iota_k = lax.broadcasted_iota(jnp.int32, (qb, _K), 1)
    bigi = jnp.int32(1 << 30)

    def step(k, cv):
        m = jnp.min(cv, axis=1, keepdims=True)
        idx = jnp.min(jnp.where(cv == m, ncand, bigi), axis=1, keepdims=True)
        idx_ref[0] = jnp.where(iota_k == k, idx + base, idx_ref[0])
        return jnp.where(ncand == idx, 1e30, cv)

    lax.fori_loop(0, _K, step, cand)


def _knn(new_xyz, xyz):
    qblk = 128
    return pl.pallas_call(
        _knn_body,
        grid=(_B, _S // qblk),
        in_specs=[
            pl.BlockSpec((1, 3, qblk), lambda b, s: (b, 0, s)),
            pl.BlockSpec((1, 3, _N), lambda b, s: (b, 0, 0)),
        ],
        out_specs=pl.BlockSpec((1, qblk, _K), lambda b, s: (b, s, 0)),
        out_shape=jax.ShapeDtypeStruct((_B, _S, _K), jnp.int32),
        scratch_shapes=[pltpu.VMEM((qblk, _K), jnp.int32)],
    )(new_xyz, xyz)


def _proj_body(xyz_ref, pts_ref, w_ref, z_ref):
    w1x = w_ref[:, 0:3]
    w1p = w_ref[:, 3:]
    zx = lax.dot_general(xyz_ref[0], w1x, (((0,), (1,)), ((), ())),
                         preferred_element_type=jnp.float32)
    zp = lax.dot_general(pts_ref[0], w1p, (((0,), (1,)), ((), ())),
                         preferred_element_type=jnp.float32)
    z_ref[0] = zx + zp


def _project(xyz, points, w1):
    nblk = 512
    return pl.pallas_call(
        _proj_body,
        grid=(_B, _N // nblk),
        in_specs=[
            pl.BlockSpec((1, 3, nblk), lambda b, n: (b, 0, n)),
            pl.BlockSpec((1, _CIN, nblk), lambda b, n: (b, 0, n)),
            pl.BlockSpec((64, _CIN + 3), lambda b, n: (0, 0)),
        ],
        out_specs=pl.BlockSpec((1, nblk, 64), lambda b, n: (b, n, 0)),
        out_shape=jax.ShapeDtypeStruct((_B, _N, 64), jnp.float32),
    )(xyz, points, w1)


def _gather_sc(table, idx):
    rows = idx.shape[0]
    d = table.shape[1]
    info = plsc.get_sparse_core_info()
    nw = info.num_cores * info.num_subcores
    chunk = 128
    per_w = rows // nw
    nchunk = per_w // chunk

    mesh = plsc.VectorSubcoreMesh(core_axis_name="c", subcore_axis_name="s")

    @functools.partial(
        pl.kernel,
        mesh=mesh,
        compiler_params=pltpu.CompilerParams(use_tc_tiling_on_sc=False),
        out_type=jax.ShapeDtypeStruct((rows, d), jnp.float32),
        scratch_types=[
            pltpu.VMEM((chunk,), jnp.int32),
            pltpu.VMEM((chunk, d), jnp.float32),
            pltpu.SemaphoreType.DMA,
        ],
    )
    def k(table_hbm, idx_hbm, out_hbm, idx_v, rows_v, sem):
        wid = lax.axis_index("s") * info.num_cores + lax.axis_index("c")
        base = wid * per_w

        def body(j, _):
            off = base + j * chunk
            pltpu.sync_copy(idx_hbm.at[pl.ds(off, chunk)], idx_v)
            pltpu.async_copy(table_hbm.at[idx_v], rows_v, sem).wait()
            pltpu.sync_copy(rows_v, out_hbm.at[pl.ds(off, chunk)])
            return 0

        lax.fori_loop(0, nchunk, body, 0)

    return k(table, idx)


def _bn_coefs(s_ref, q_ref, g_ref, b_ref):
    mean = s_ref[...] / _M
    var = q_ref[...] / _M - mean * mean
    scale = g_ref[...] / jnp.sqrt(var + _EPS)
    shift = b_ref[...] - mean * scale
    return scale, shift


def _acc_stats(first, y, s_ref, q_ref, width):
    psum = jnp.sum(y, axis=0).reshape(1, width)
    pq = jnp.sum(y * y, axis=0).reshape(1, width)

    @pl.when(first)
    def _():
        s_ref[...] = jnp.zeros_like(s_ref)
        q_ref[...] = jnp.zeros_like(q_ref)

    s_ref[...] += psum
    q_ref[...] += pq


def _first(b, sb):
    return jnp.logical_and(b == 0, sb == 0)


def _stats1_body(zg_ref, nx_ref, w_ref, c1_ref, s_ref, q_ref):
    w1x = w_ref[:, 0:3]
    c1 = lax.dot_general(nx_ref[0], w1x, (((0,), (1,)), ((), ())),
                         preferred_element_type=jnp.float32)
    c1_ref[0] = c1
    sblk = c1.shape[0]
    zg = zg_ref[0].reshape(sblk, _K, 64)
    y1 = (zg - c1[:, None, :]).reshape(sblk * _K, 64)
    _acc_stats(_first(pl.program_id(0), pl.program_id(1)), y1, s_ref, q_ref, 64)


def _stats1(zg3, new_xyz, w1):
    sblk = 128
    return pl.pallas_call(
        _stats1_body,
        grid=(_B, _S // sblk),
        in_specs=[
            pl.BlockSpec((1, sblk * _K, 64), lambda b, s: (b, s, 0)),
            pl.BlockSpec((1, 3, sblk), lambda b, s: (b, 0, s)),
            pl.BlockSpec((64, _CIN + 3), lambda b, s: (0, 0)),
        ],
        out_specs=[
            pl.BlockSpec((1, sblk, 64), lambda b, s: (b, s, 0)),
            pl.BlockSpec((1, 64), lambda b, s: (0, 0)),
            pl.BlockSpec((1, 64), lambda b, s: (0, 0)),
        ],
        out_shape=[
            jax.ShapeDtypeStruct((_B, _S, 64), jnp.float32),
            jax.ShapeDtypeStruct((1, 64), jnp.float32),
            jax.ShapeDtypeStruct((1, 64), jnp.float32),
        ],
    )(zg3, new_xyz, w1)


def _y2_of(zg_ref, c1_ref, s1_ref, q1_ref, g1_ref, b1_ref, w2_ref):
    scale, shift = _bn_coefs(s1_ref, q1_ref, g1_ref, b1_ref)
    c1 = c1_ref[0]
    sblk = c1.shape[0]
    zg = zg_ref[0].reshape(sblk, _K, 64)
    y1 = zg - c1[:, None, :]
    y1n = jnp.maximum(y1 * scale.reshape(1, 1, 64) + shift.reshape(1, 1, 64), 0.0)
    return lax.dot_general(y1n.reshape(sblk * _K, 64), w2_ref[...],
                           (((1,), (1,)), ((), ())),
                           preferred_element_type=jnp.float32)


def _mlp_specs(sblk, extra):
    return [
        pl.BlockSpec((1, sblk * _K, 64), lambda b, s: (b, s, 0)),
        pl.BlockSpec((1, sblk, 64), lambda b, s: (b, s, 0)),
        pl.BlockSpec((1, 64), lambda b, s: (0, 0)),
        pl.BlockSpec((1, 64), lambda b, s: (0, 0)),
        pl.BlockSpec((1, 64), lambda b, s: (0, 0)),
        pl.BlockSpec((1, 64), lambda b, s: (0, 0)),
        pl.BlockSpec((128, 64), lambda b, s: (0, 0)),
    ] + extra


def _l2s_body(zg_ref, c1_ref, s1_ref, q1_ref, g1_ref, b1_ref, w2_ref,
              s2_ref, q2_ref):
    y2 = _y2_of(zg_ref, c1_ref, s1_ref, q1_ref, g1_ref, b1_ref, w2_ref)
    _acc_stats(_first(pl.program_id(0), pl.program_id(1)), y2, s2_ref, q2_ref, 128)


def _l2_stats(zg3, c1, s1, q1, g1, b1, w2):
    sblk = 32
    return pl.pallas_call(
        _l2s_body,
        grid=(_B, _S // sblk),
        in_specs=_mlp_specs(sblk, []),
        out_specs=[
            pl.BlockSpec((1, 128), lambda b, s: (0, 0)),
            pl.BlockSpec((1, 128), lambda b, s: (0, 0)),
        ],
        out_shape=[
            jax.ShapeDtypeStruct((1, 128), jnp.float32),
            jax.ShapeDtypeStruct((1, 128), jnp.float32),
        ],
    )(zg3, c1, s1, q1, g1, b1, w2)


def _l3s_body(zg_ref, c1_ref, s1_ref, q1_ref, g1_ref, b1_ref, w2_ref,
              s2_ref, q2_ref, g2_ref, b2_ref, w3_ref, s3_ref, q3_ref):
    y2 = _y2_of(zg_ref, c1_ref, s1_ref, q1_ref, g1_ref, b1_ref, w2_ref)
    scale2, shift2 = _bn_coefs(s2_ref, q2_ref, g2_ref, b2_ref)
    y2n = jnp.maximum(y2 * scale2 + shift2, 0.0)
    y3 = lax.dot_general(y2n, w3_ref[...], (((1,), (1,)), ((), ())),
                         preferred_element_type=jnp.float32)
    _acc_stats(_first(pl.program_id(0), pl.program_id(1)), y3, s3_ref, q3_ref, 256)


def _l3_stats(zg3, c1, s1, q1, g1, b1, w2, s2, q2, g2, b2, w3):
    sblk = 32
    extra = [
        pl.BlockSpec((1, 128), lambda b, s: (0, 0)),
        pl.BlockSpec((1, 128), lambda b, s: (0, 0)),
        pl.BlockSpec((1, 128), lambda b, s: (0, 0)),
        pl.BlockSpec((1, 128), lambda b, s: (0, 0)),
        pl.BlockSpec((256, 128), lambda b, s: (0, 0)),
    ]
    return pl.pallas_call(
        _l3s_body,
        grid=(_B, _S // sblk),
        in_specs=_mlp_specs(sblk, extra),
        out_specs=[
            pl.BlockSpec((1, 256), lambda b, s: (0, 0)),
            pl.BlockSpec((1, 256), lambda b, s: (0, 0)),
        ],
        out_shape=[
            jax.ShapeDtypeStruct((1, 256), jnp.float32),
            jax.ShapeDtypeStruct((1, 256), jnp.float32),
        ],
    )(zg3, c1, s1, q1, g1, b1, w2, s2, q2, g2, b2, w3)


def _final_body(zg_ref, c1_ref, s1_ref, q1_ref, g1_ref, b1_ref, w2_ref,
                s2_ref, q2_ref, g2_ref, b2_ref, w3_ref,
                s3_ref, q3_ref, g3_ref, b3_ref, out_ref):
    y2 = _y2_of(zg_ref, c1_ref, s1_ref, q1_ref, g1_ref, b1_ref, w2_ref)
    scale2, shift2 = _bn_coefs(s2_ref, q2_ref, g2_ref, b2_ref)
    y2n = jnp.maximum(y2 * scale2 + shift2, 0.0)
    y3 = lax.dot_general(y2n, w3_ref[...], (((1,), (1,)), ((), ())),
                         preferred_element_type=jnp.float32)
    scale3, shift3 = _bn_coefs(s3_ref, q3_ref, g3_ref, b3_ref)
    y3n = jnp.maximum(y3 * scale3 + shift3, 0.0)
    sblk = y3n.shape[0] // _K
    out_ref[0] = jnp.max(y3n.reshape(sblk, _K, 256), axis=1)


def _final(zg3, c1, s1, q1, g1, b1, w2, s2, q2, g2, b2, w3, s3, q3, g3, b3):
    sblk = 32
    extra = [
        pl.BlockSpec((1, 128), lambda b, s: (0, 0)),
        pl.BlockSpec((1, 128), lambda b, s: (0, 0)),
        pl.BlockSpec((1, 128), lambda b, s: (0, 0)),
        pl.BlockSpec((1, 128), lambda b, s: (0, 0)),
        pl.BlockSpec((256, 128), lambda b, s: (0, 0)),
        pl.BlockSpec((1, 256), lambda b, s: (0, 0)),
        pl.BlockSpec((1, 256), lambda b, s: (0, 0)),
        pl.BlockSpec((1, 256), lambda b, s: (0, 0)),
        pl.BlockSpec((1, 256), lambda b, s: (0, 0)),
    ]
    return pl.pallas_call(
        _final_body,
        grid=(_B, _S // sblk),
        in_specs=_mlp_specs(sblk, extra),
        out_specs=pl.BlockSpec((1, sblk, 256), lambda b, s: (b, s, 0)),
        out_shape=jax.ShapeDtypeStruct((_B, _S, 256), jnp.float32),
    )(zg3, c1, s1, q1, g1, b1, w2, s2, q2, g2, b2, w3, s3, q3, g3, b3)


def kernel(xyz, points, W1, g1, b1, W2, g2, b2, W3, g3, b3):
    new_xyz = _fps(xyz)
    z = _project(xyz, points, W1)
    flat_idx = _knn(new_xyz, xyz).reshape(-1)
    zg = _gather_sc(z.reshape(_B * _N, 64), flat_idx)
    zg3 = zg.reshape(_B, _S * _K, 64)

    g1r, b1r = g1.reshape(1, 64), b1.reshape(1, 64)
    g2r, b2r = g2.reshape(1, 128), b2.reshape(1, 128)
    g3r, b3r = g3.reshape(1, 256), b3.reshape(1, 256)

    c1, s1, q1 = _stats1(zg3, new_xyz, W1)
    s2, q2 = _l2_stats(zg3, c1, s1, q1, g1r, b1r, W2)
    s3, q3 = _l3_stats(zg3, c1, s1, q1, g1r, b1r, W2, s2, q2, g2r, b2r, W3)
    out = _final(zg3, c1, s1, q1, g1r, b1r, W2, s2, q2, g2r, b2r, W3,
                 s3, q3, g3r, b3r)

    return (new_xyz, jnp.transpose(out, (0, 2, 1)))
